# baseline (speedup 1.0000x reference)
import functools

import jax
import jax.numpy as jnp
from jax import lax
from jax.experimental import pallas as pl
from jax.experimental.pallas import tpu as pltpu
from jax.experimental.pallas import tpu_sc as plsc

N = 512
D = 256
K = 10000
KN = 32
TEMP = 0.2
NEG_TOTAL = N * KN


def _sc_meta(idx, centroids, label_bank, loss_weight):
  info = plsc.get_sparse_core_info()
  nc, ns = info.num_cores, info.num_subcores
  nw = nc * ns
  b_per_w = N // nw

  mesh = plsc.VectorSubcoreMesh(core_axis_name="c", subcore_axis_name="s")

  @functools.partial(
      pl.kernel,
      out_type=(
          jax.ShapeDtypeStruct((N,), jnp.int32),
          jax.ShapeDtypeStruct((N,), jnp.float32),
          jax.ShapeDtypeStruct((N, D), jnp.float32),
      ),
      mesh=mesh,
      scratch_types=[
          pltpu.VMEM((b_per_w,), jnp.int32),
          pltpu.VMEM((b_per_w,), jnp.int32),
          pltpu.VMEM((b_per_w,), jnp.float32),
          pltpu.VMEM((b_per_w, D), jnp.float32),
          pltpu.SemaphoreType.DMA,
      ],
  )
  def sc_kernel(idx_hbm, cent_hbm, lbank_hbm, lw_hbm,
                labels_out, w_out, pos_out,
                idx_v, labels_v, w_v, pos_v, sem):
    wid = lax.axis_index("s") * nc + lax.axis_index("c")
    base = wid * b_per_w
    pltpu.sync_copy(idx_hbm.at[pl.ds(base, b_per_w)], idx_v)
    pltpu.async_copy(lbank_hbm.at[idx_v], labels_v, sem).wait()
    pltpu.sync_copy(labels_v, labels_out.at[pl.ds(base, b_per_w)])
    pltpu.async_copy(lw_hbm.at[labels_v], w_v, sem).wait()
    pltpu.sync_copy(w_v, w_out.at[pl.ds(base, b_per_w)])
    pltpu.async_copy(cent_hbm.at[labels_v], pos_v, sem).wait()
    pltpu.sync_copy(pos_v, pos_out.at[pl.ds(base, b_per_w)])

  return sc_kernel(idx, centroids, label_bank, loss_weight)


def _sc_neg(neg_idx_flat, centroids):
  info = plsc.get_sparse_core_info()
  nc, ns = info.num_cores, info.num_subcores
  nw = nc * ns
  neg_per_w = NEG_TOTAL // nw
  nchunk = 128
  nloops = neg_per_w // nchunk

  mesh = plsc.VectorSubcoreMesh(core_axis_name="c", subcore_axis_name="s")

  @functools.partial(
      pl.kernel,
      out_type=jax.ShapeDtypeStruct((NEG_TOTAL, D), jnp.float32),
      mesh=mesh,
      scratch_types=[
          pltpu.VMEM((neg_per_w,), jnp.int32),
          pltpu.VMEM((nchunk, D), jnp.float32),
          pltpu.VMEM((nchunk, D), jnp.float32),
          pltpu.SemaphoreType.DMA,
          pltpu.SemaphoreType.DMA,
      ],
  )
  def sc_kernel(negidx_hbm, cent_hbm, neg_out,
                nidx_v, rows_a, rows_b, sem_a, sem_b):
    wid = lax.axis_index("s") * nc + lax.axis_index("c")
    nbase = wid * neg_per_w
    pltpu.sync_copy(negidx_hbm.at[pl.ds(nbase, neg_per_w)], nidx_v)
    bufs = (rows_a, rows_b)
    sems = (sem_a, sem_b)
    copies = [None, None]
    for c in range(nloops):
      b = c % 2
      copies[b] = pltpu.async_copy(
          cent_hbm.at[nidx_v.at[pl.ds(c * nchunk, nchunk)]], bufs[b], sems[b])
      if c >= 1:
        pb = (c - 1) % 2
        copies[pb].wait()
        pltpu.sync_copy(bufs[pb],
                        neg_out.at[pl.ds(nbase + (c - 1) * nchunk, nchunk)])
    last = (nloops - 1) % 2
    copies[last].wait()
    pltpu.sync_copy(bufs[last],
                    neg_out.at[pl.ds(nbase + (nloops - 1) * nchunk, nchunk)])

  return sc_kernel(neg_idx_flat, centroids)


_BK = 1024
_G = 10
_KP = _G * _BK


def _tc1_body(z_ref, fo_ref, lab_ref, w_ref, hw_ref, hb_ref, out_ref,
              s_sc, lab_sc, ins_sc, fo_bf_sc):
  pid = pl.program_id(0)

  @pl.when(pid == 0)
  def _():
    s_sc[...] = jnp.zeros((N, 1), jnp.float32)
    lab_sc[...] = jnp.zeros((N, 1), jnp.float32)
    fo_bf_sc[...] = fo_ref[...].astype(jnp.bfloat16)
    m = 2 * N
    z = z_ref[...]
    zn = z / (jnp.sqrt(jnp.sum(z * z, axis=1, keepdims=True)) + 1e-10)
    znb = zn.astype(jnp.bfloat16)
    s = lax.dot_general(znb, znb, (((1,), (1,)), ((), ())),
                        preferred_element_type=jnp.float32)
    r = lax.broadcasted_iota(jnp.int32, (m, m), 0)
    c = lax.broadcasted_iota(jnp.int32, (m, m), 1)
    sm = jnp.where(r == c, jnp.float32(-1e30), s * (1.0 / TEMP))
    lse = jnp.log(jnp.sum(jnp.exp(sm), axis=1, keepdims=True))
    pos = jnp.sum(jnp.where((r ^ 1) == c, s, 0.0), axis=1, keepdims=True)
    ins_sc[...] = jnp.reshape(jnp.sum(lse - pos * (1.0 / TEMP)) / m, (1, 1))

  scores = jnp.dot(fo_bf_sc[...], hw_ref[...],
                   preferred_element_type=jnp.float32) + hb_ref[...]
  col = pid * _BK + lax.broadcasted_iota(jnp.int32, (N, _BK), 1)
  labm = col == lab_ref[...]
  lab_sc[...] = lab_sc[...] + jnp.sum(
      jnp.where(labm, scores, 0.0), axis=1, keepdims=True)
  s_sc[...] = s_sc[...] + jnp.sum(jnp.exp(scores), axis=1, keepdims=True)

  @pl.when(pid == _G - 1)
  def _():
    lse = jnp.log(s_sc[...])
    nll = lse - lab_sc[...]
    wv = w_ref[...]
    l_cls = jnp.sum(wv * nll) / jnp.sum(wv)
    out_ref[...] = ins_sc[...] + jnp.reshape(l_cls, (1, 1))


def _tc1(z, f_odc, labels, w, hw_bf_pad, hb_pad, interpret=False):
  return pl.pallas_call(
      _tc1_body,
      grid=(_G,),
      in_specs=[
          pl.BlockSpec((2 * N, D), lambda i: (0, 0)),
          pl.BlockSpec((N, D), lambda i: (0, 0)),
          pl.BlockSpec((N, 1), lambda i: (0, 0)),
          pl.BlockSpec((N, 1), lambda i: (0, 0)),
          pl.BlockSpec((D, _BK), lambda i: (0, i)),
          pl.BlockSpec((1, _BK), lambda i: (0, i)),
      ],
      out_specs=pl.BlockSpec((1, 1), lambda i: (0, 0)),
      out_shape=jax.ShapeDtypeStruct((1, 1), jnp.float32),
      scratch_shapes=[
          pltpu.VMEM((N, 1), jnp.float32),
          pltpu.VMEM((N, 1), jnp.float32),
          pltpu.VMEM((1, 1), jnp.float32),
          pltpu.VMEM((N, D), jnp.bfloat16),
      ],
      interpret=interpret,
  )(z, f_odc, labels, w, hw_bf_pad, hb_pad)


_CCH = 128
_CG = N // _CCH


def _tc2_body(fo_ref, fc_ref, pos_ref, neg_ref, p1_ref, out_ref, acc_sc):
  pid = pl.program_id(0)

  @pl.when(pid == 0)
  def _():
    acc_sc[...] = jnp.zeros((1, 1), jnp.float32)

  fo = fo_ref[...]
  fc = fc_ref[...]
  pc = pos_ref[...]
  fon = fo / jnp.sqrt(jnp.sum(fo * fo, axis=1, keepdims=True))
  fcn = fc / jnp.sqrt(jnp.sum(fc * fc, axis=1, keepdims=True))
  pcn = pc / jnp.sqrt(jnp.sum(pc * pc, axis=1, keepdims=True))
  p1 = jnp.sum(fon * pcn, axis=1, keepdims=True) * (1.0 / TEMP)
  p2 = jnp.sum(fcn * pcn, axis=1, keepdims=True) * (1.0 / TEMP)

  ncg = jnp.reshape(neg_ref[...], (_CCH, KN, D))
  ncb = ncg.astype(jnp.bfloat16)
  nss = jnp.sum(ncg * ncg, axis=2)
  inv = lax.rsqrt(nss)
  fon_b = fon.astype(jnp.bfloat16)
  fcn_b = fcn.astype(jnp.bfloat16)
  d1 = jnp.sum((fon_b[:, None, :] * ncb).astype(jnp.float32),
               axis=2) * inv * (1.0 / TEMP)
  d2 = jnp.sum((fcn_b[:, None, :] * ncb).astype(jnp.float32),
               axis=2) * inv * (1.0 / TEMP)
  lse1 = jnp.log(jnp.exp(p1) + jnp.sum(jnp.exp(d1), axis=1, keepdims=True))
  lse2 = jnp.log(jnp.exp(p2) + jnp.sum(jnp.exp(d2), axis=1, keepdims=True))
  acc_sc[...] = acc_sc[...] + jnp.reshape(
      jnp.sum(lse1 - p1) + jnp.sum(lse2 - p2), (1, 1))

  @pl.when(pid == _CG - 1)
  def _():
    out_ref[...] = p1_ref[...] + acc_sc[...] * (1.0 / N)


def _tc2(f_odc, f_cts, pos_c, neg_c, part1, interpret=False):
  return pl.pallas_call(
      _tc2_body,
      grid=(_CG,),
      in_specs=[
          pl.BlockSpec((_CCH, D), lambda i: (i, 0)),
          pl.BlockSpec((_CCH, D), lambda i: (i, 0)),
          pl.BlockSpec((_CCH, D), lambda i: (i, 0)),
          pl.BlockSpec((_CCH * KN, D), lambda i: (i, 0)),
          pl.BlockSpec((1, 1), lambda i: (0, 0)),
      ],
      out_specs=pl.BlockSpec((1, 1), lambda i: (0, 0)),
      out_shape=jax.ShapeDtypeStruct((1, 1), jnp.float32),
      scratch_shapes=[
          pltpu.VMEM((1, 1), jnp.float32),
      ],
      interpret=interpret,
  )(f_odc, f_cts, pos_c, neg_c, part1)


def kernel(z, idx, neg_idx, centroids, label_bank, head_w, head_b, loss_weight):
  labels, w, pos_c = _sc_meta(
      idx.astype(jnp.int32), centroids, label_bank.astype(jnp.int32),
      loss_weight)
  neg_c = _sc_neg(neg_idx.reshape(-1).astype(jnp.int32), centroids)
  f_odc = z[0::2]
  f_cts = z[1::2]
  hw_bf = jnp.pad(head_w.astype(jnp.bfloat16), ((0, 0), (0, _KP - K)))
  hb_pad = jnp.pad(head_b, (0, _KP - K),
                   constant_values=-1e30).reshape(1, _KP)
  part1 = _tc1(z, f_odc, labels.reshape(N, 1), w.reshape(N, 1),
               hw_bf, hb_pad)
  out = _tc2(f_odc, f_cts, pos_c, neg_c, part1)
  return out[0, 0]


# speedup vs baseline: 1.0712x; 1.0712x over previous
import functools

import jax
import jax.numpy as jnp
from jax import lax
from jax.experimental import pallas as pl
from jax.experimental.pallas import tpu as pltpu
from jax.experimental.pallas import tpu_sc as plsc

N = 512
D = 256
K = 10000
KN = 32
TEMP = 0.2
NEG_TOTAL = N * KN


def _sc_meta(idx, centroids, label_bank, loss_weight):
  info = plsc.get_sparse_core_info()
  nc, ns = info.num_cores, info.num_subcores
  nw = nc * ns
  b_per_w = N // nw

  mesh = plsc.VectorSubcoreMesh(core_axis_name="c", subcore_axis_name="s")

  @functools.partial(
      pl.kernel,
      out_type=(
          jax.ShapeDtypeStruct((N,), jnp.int32),
          jax.ShapeDtypeStruct((N,), jnp.float32),
          jax.ShapeDtypeStruct((N, D), jnp.float32),
      ),
      mesh=mesh,
      scratch_types=[
          pltpu.VMEM((b_per_w,), jnp.int32),
          pltpu.VMEM((b_per_w,), jnp.int32),
          pltpu.VMEM((b_per_w,), jnp.float32),
          pltpu.VMEM((b_per_w, D), jnp.float32),
          pltpu.SemaphoreType.DMA,
      ],
  )
  def sc_kernel(idx_hbm, cent_hbm, lbank_hbm, lw_hbm,
                labels_out, w_out, pos_out,
                idx_v, labels_v, w_v, pos_v, sem):
    wid = lax.axis_index("s") * nc + lax.axis_index("c")
    base = wid * b_per_w
    pltpu.sync_copy(idx_hbm.at[pl.ds(base, b_per_w)], idx_v)
    pltpu.async_copy(lbank_hbm.at[idx_v], labels_v, sem).wait()
    pltpu.sync_copy(labels_v, labels_out.at[pl.ds(base, b_per_w)])
    pltpu.async_copy(lw_hbm.at[labels_v], w_v, sem).wait()
    pltpu.sync_copy(w_v, w_out.at[pl.ds(base, b_per_w)])
    pltpu.async_copy(cent_hbm.at[labels_v], pos_v, sem).wait()
    pltpu.sync_copy(pos_v, pos_out.at[pl.ds(base, b_per_w)])

  return sc_kernel(idx, centroids, label_bank, loss_weight)


def _sc_neg(neg_idx_flat, centroids):
  info = plsc.get_sparse_core_info()
  nc, ns = info.num_cores, info.num_subcores
  nw = nc * ns
  neg_per_w = NEG_TOTAL // nw
  nchunk = 128
  nloops = neg_per_w // nchunk

  mesh = plsc.VectorSubcoreMesh(core_axis_name="c", subcore_axis_name="s")

  @functools.partial(
      pl.kernel,
      out_type=jax.ShapeDtypeStruct((NEG_TOTAL, D), jnp.float32),
      mesh=mesh,
      scratch_types=[
          pltpu.VMEM((neg_per_w,), jnp.int32),
          pltpu.VMEM((nchunk, D), jnp.float32),
          pltpu.VMEM((nchunk, D), jnp.float32),
          pltpu.SemaphoreType.DMA,
          pltpu.SemaphoreType.DMA,
      ],
  )
  def sc_kernel(negidx_hbm, cent_hbm, neg_out,
                nidx_v, rows_a, rows_b, sem_a, sem_b):
    wid = lax.axis_index("s") * nc + lax.axis_index("c")
    nbase = wid * neg_per_w
    pltpu.sync_copy(negidx_hbm.at[pl.ds(nbase, neg_per_w)], nidx_v)
    bufs = (rows_a, rows_b)
    sems = (sem_a, sem_b)
    copies = [None, None]
    for c in range(nloops):
      b = c % 2
      copies[b] = pltpu.async_copy(
          cent_hbm.at[nidx_v.at[pl.ds(c * nchunk, nchunk)]], bufs[b], sems[b])
      if c >= 1:
        pb = (c - 1) % 2
        copies[pb].wait()
        pltpu.sync_copy(bufs[pb],
                        neg_out.at[pl.ds(nbase + (c - 1) * nchunk, nchunk)])
    last = (nloops - 1) % 2
    copies[last].wait()
    pltpu.sync_copy(bufs[last],
                    neg_out.at[pl.ds(nbase + (nloops - 1) * nchunk, nchunk)])

  return sc_kernel(neg_idx_flat, centroids)


_BK = 1024
_G = 9
_KT = K - _G * _BK


def _tc1_body(z_ref, fo_ref, lab_ref, w_ref, hw_ref, hb_ref,
              hwt_ref, hbt_ref, out_ref,
              s_sc, lab_sc, ins_sc, fo_bf_sc):
  pid = pl.program_id(0)

  @pl.when(pid == 0)
  def _():
    s_sc[...] = jnp.zeros((N, 1), jnp.float32)
    lab_sc[...] = jnp.zeros((N, 1), jnp.float32)
    fo_bf_sc[...] = fo_ref[...].astype(jnp.bfloat16)
    m = 2 * N
    z = z_ref[...]
    zn = z / (jnp.sqrt(jnp.sum(z * z, axis=1, keepdims=True)) + 1e-10)
    znb = zn.astype(jnp.bfloat16)
    s = lax.dot_general(znb, znb, (((1,), (1,)), ((), ())),
                        preferred_element_type=jnp.float32)
    r = lax.broadcasted_iota(jnp.int32, (m, m), 0)
    c = lax.broadcasted_iota(jnp.int32, (m, m), 1)
    sm = jnp.where(r == c, jnp.float32(-1e30), s * (1.0 / TEMP))
    lse = jnp.log(jnp.sum(jnp.exp(sm), axis=1, keepdims=True))
    pos = jnp.sum(jnp.where((r ^ 1) == c, s, 0.0), axis=1, keepdims=True)
    ins_sc[...] = jnp.reshape(jnp.sum(lse - pos * (1.0 / TEMP)) / m, (1, 1))

  scores = jnp.dot(fo_bf_sc[...], hw_ref[...],
                   preferred_element_type=jnp.float32) + hb_ref[...]
  col = pid * _BK + lax.broadcasted_iota(jnp.int32, (N, _BK), 1)
  labm = col == lab_ref[...]
  lab_sc[...] = lab_sc[...] + jnp.sum(
      jnp.where(labm, scores, 0.0), axis=1, keepdims=True)
  s_sc[...] = s_sc[...] + jnp.sum(jnp.exp(scores), axis=1, keepdims=True)

  @pl.when(pid == _G - 1)
  def _():
    scores_t = jnp.dot(fo_bf_sc[...], hwt_ref[...],
                       preferred_element_type=jnp.float32) + hbt_ref[...]
    col_t = _G * _BK + lax.broadcasted_iota(jnp.int32, (N, _KT), 1)
    labm_t = col_t == lab_ref[...]
    lab_tot = lab_sc[...] + jnp.sum(
        jnp.where(labm_t, scores_t, 0.0), axis=1, keepdims=True)
    s_tot = s_sc[...] + jnp.sum(jnp.exp(scores_t), axis=1, keepdims=True)
    lse = jnp.log(s_tot)
    nll = lse - lab_tot
    wv = w_ref[...]
    l_cls = jnp.sum(wv * nll) / jnp.sum(wv)
    out_ref[...] = ins_sc[...] + jnp.reshape(l_cls, (1, 1))


def _tc1(z, f_odc, labels, w, hw_bf, hb2, hwt_bf, hbt, interpret=False):
  return pl.pallas_call(
      _tc1_body,
      grid=(_G,),
      in_specs=[
          pl.BlockSpec((2 * N, D), lambda i: (0, 0)),
          pl.BlockSpec((N, D), lambda i: (0, 0)),
          pl.BlockSpec((N, 1), lambda i: (0, 0)),
          pl.BlockSpec((N, 1), lambda i: (0, 0)),
          pl.BlockSpec((D, _BK), lambda i: (0, i)),
          pl.BlockSpec((1, _BK), lambda i: (0, i)),
          pl.BlockSpec((D, _KT), lambda i: (0, 0)),
          pl.BlockSpec((1, _KT), lambda i: (0, 0)),
      ],
      out_specs=pl.BlockSpec((1, 1), lambda i: (0, 0)),
      out_shape=jax.ShapeDtypeStruct((1, 1), jnp.float32),
      scratch_shapes=[
          pltpu.VMEM((N, 1), jnp.float32),
          pltpu.VMEM((N, 1), jnp.float32),
          pltpu.VMEM((1, 1), jnp.float32),
          pltpu.VMEM((N, D), jnp.bfloat16),
      ],
      interpret=interpret,
  )(z, f_odc, labels, w, hw_bf, hb2, hwt_bf, hbt)


_CCH = 128
_CG = N // _CCH


def _tc2_body(fo_ref, fc_ref, pos_ref, neg_ref, p1_ref, out_ref, acc_sc):
  pid = pl.program_id(0)

  @pl.when(pid == 0)
  def _():
    acc_sc[...] = jnp.zeros((1, 1), jnp.float32)

  fo = fo_ref[...]
  fc = fc_ref[...]
  pc = pos_ref[...]
  fon = fo / jnp.sqrt(jnp.sum(fo * fo, axis=1, keepdims=True))
  fcn = fc / jnp.sqrt(jnp.sum(fc * fc, axis=1, keepdims=True))
  pcn = pc / jnp.sqrt(jnp.sum(pc * pc, axis=1, keepdims=True))
  p1 = jnp.sum(fon * pcn, axis=1, keepdims=True) * (1.0 / TEMP)
  p2 = jnp.sum(fcn * pcn, axis=1, keepdims=True) * (1.0 / TEMP)

  ncg = jnp.reshape(neg_ref[...], (_CCH, KN, D))
  ncb = ncg.astype(jnp.bfloat16)
  nss = jnp.sum(ncg * ncg, axis=2)
  inv = lax.rsqrt(nss)
  fon_b = fon.astype(jnp.bfloat16)
  fcn_b = fcn.astype(jnp.bfloat16)
  d1 = jnp.sum((fon_b[:, None, :] * ncb).astype(jnp.float32),
               axis=2) * inv * (1.0 / TEMP)
  d2 = jnp.sum((fcn_b[:, None, :] * ncb).astype(jnp.float32),
               axis=2) * inv * (1.0 / TEMP)
  lse1 = jnp.log(jnp.exp(p1) + jnp.sum(jnp.exp(d1), axis=1, keepdims=True))
  lse2 = jnp.log(jnp.exp(p2) + jnp.sum(jnp.exp(d2), axis=1, keepdims=True))
  acc_sc[...] = acc_sc[...] + jnp.reshape(
      jnp.sum(lse1 - p1) + jnp.sum(lse2 - p2), (1, 1))

  @pl.when(pid == _CG - 1)
  def _():
    out_ref[...] = p1_ref[...] + acc_sc[...] * (1.0 / N)


def _tc2(f_odc, f_cts, pos_c, neg_c, part1, interpret=False):
  return pl.pallas_call(
      _tc2_body,
      grid=(_CG,),
      in_specs=[
          pl.BlockSpec((_CCH, D), lambda i: (i, 0)),
          pl.BlockSpec((_CCH, D), lambda i: (i, 0)),
          pl.BlockSpec((_CCH, D), lambda i: (i, 0)),
          pl.BlockSpec((_CCH * KN, D), lambda i: (i, 0)),
          pl.BlockSpec((1, 1), lambda i: (0, 0)),
      ],
      out_specs=pl.BlockSpec((1, 1), lambda i: (0, 0)),
      out_shape=jax.ShapeDtypeStruct((1, 1), jnp.float32),
      scratch_shapes=[
          pltpu.VMEM((1, 1), jnp.float32),
      ],
      interpret=interpret,
  )(f_odc, f_cts, pos_c, neg_c, part1)


def kernel(z, idx, neg_idx, centroids, label_bank, head_w, head_b, loss_weight):
  labels, w, pos_c = _sc_meta(
      idx.astype(jnp.int32), centroids, label_bank.astype(jnp.int32),
      loss_weight)
  neg_c = _sc_neg(neg_idx.reshape(-1).astype(jnp.int32), centroids)
  f_odc = z[0::2]
  f_cts = z[1::2]
  hw_bf = head_w.astype(jnp.bfloat16)
  hb2 = head_b.reshape(1, K)
  sp = _G * _BK
  part1 = _tc1(z, f_odc, labels.reshape(N, 1), w.reshape(N, 1),
               hw_bf[:, :sp], hb2[:, :sp], hw_bf[:, sp:], hb2[:, sp:])
  out = _tc2(f_odc, f_cts, pos_c, neg_c, part1)
  return out[0, 0]


# speedup vs baseline: 1.1307x; 1.0555x over previous
import functools

import jax
import jax.numpy as jnp
from jax import lax
from jax.experimental import pallas as pl
from jax.experimental.pallas import tpu as pltpu
from jax.experimental.pallas import tpu_sc as plsc

N = 512
D = 256
K = 10000
KN = 32
TEMP = 0.2
NEG_TOTAL = N * KN


def _sc_meta(idx, centroids, label_bank, loss_weight):
  info = plsc.get_sparse_core_info()
  nc, ns = info.num_cores, info.num_subcores
  nw = nc * ns
  b_per_w = N // nw

  mesh = plsc.VectorSubcoreMesh(core_axis_name="c", subcore_axis_name="s")

  @functools.partial(
      pl.kernel,
      out_type=(
          jax.ShapeDtypeStruct((N,), jnp.int32),
          jax.ShapeDtypeStruct((N,), jnp.float32),
          jax.ShapeDtypeStruct((N, D), jnp.float32),
      ),
      mesh=mesh,
      scratch_types=[
          pltpu.VMEM((b_per_w,), jnp.int32),
          pltpu.VMEM((b_per_w,), jnp.int32),
          pltpu.VMEM((b_per_w,), jnp.float32),
          pltpu.VMEM((b_per_w, D), jnp.float32),
          pltpu.SemaphoreType.DMA,
      ],
  )
  def sc_kernel(idx_hbm, cent_hbm, lbank_hbm, lw_hbm,
                labels_out, w_out, pos_out,
                idx_v, labels_v, w_v, pos_v, sem):
    wid = lax.axis_index("s") * nc + lax.axis_index("c")
    base = wid * b_per_w
    pltpu.sync_copy(idx_hbm.at[pl.ds(base, b_per_w)], idx_v)
    pltpu.async_copy(lbank_hbm.at[idx_v], labels_v, sem).wait()
    pltpu.sync_copy(labels_v, labels_out.at[pl.ds(base, b_per_w)])
    pltpu.async_copy(lw_hbm.at[labels_v], w_v, sem).wait()
    pltpu.sync_copy(w_v, w_out.at[pl.ds(base, b_per_w)])
    pltpu.async_copy(cent_hbm.at[labels_v], pos_v, sem).wait()
    pltpu.sync_copy(pos_v, pos_out.at[pl.ds(base, b_per_w)])

  return sc_kernel(idx, centroids, label_bank, loss_weight)


def _sc_neg(neg_idx_flat, centroids):
  info = plsc.get_sparse_core_info()
  nc, ns = info.num_cores, info.num_subcores
  nw = nc * ns
  neg_per_w = NEG_TOTAL // nw
  nchunk = 128
  nloops = neg_per_w // nchunk

  mesh = plsc.VectorSubcoreMesh(core_axis_name="c", subcore_axis_name="s")

  @functools.partial(
      pl.kernel,
      out_type=jax.ShapeDtypeStruct((NEG_TOTAL, D), jnp.float32),
      mesh=mesh,
      scratch_types=[
          pltpu.VMEM((neg_per_w,), jnp.int32),
          pltpu.VMEM((nchunk, D), jnp.float32),
          pltpu.VMEM((nchunk, D), jnp.float32),
          pltpu.SemaphoreType.DMA,
          pltpu.SemaphoreType.DMA,
      ],
  )
  def sc_kernel(negidx_hbm, cent_hbm, neg_out,
                nidx_v, rows_a, rows_b, sem_a, sem_b):
    wid = lax.axis_index("s") * nc + lax.axis_index("c")
    nbase = wid * neg_per_w
    pltpu.sync_copy(negidx_hbm.at[pl.ds(nbase, neg_per_w)], nidx_v)
    bufs = (rows_a, rows_b)
    sems = (sem_a, sem_b)
    copies = [None, None]
    for c in range(nloops):
      b = c % 2
      copies[b] = pltpu.async_copy(
          cent_hbm.at[nidx_v.at[pl.ds(c * nchunk, nchunk)]], bufs[b], sems[b])
      if c >= 1:
        pb = (c - 1) % 2
        copies[pb].wait()
        pltpu.sync_copy(bufs[pb],
                        neg_out.at[pl.ds(nbase + (c - 1) * nchunk, nchunk)])
    last = (nloops - 1) % 2
    copies[last].wait()
    pltpu.sync_copy(bufs[last],
                    neg_out.at[pl.ds(nbase + (nloops - 1) * nchunk, nchunk)])

  return sc_kernel(neg_idx_flat, centroids)


_BK = 1024
_NFULL = K // _BK
_KT = K - _NFULL * _BK


def _tc1_body(z_ref, fo_ref, lab_ref, w_ref, hw_ref, hb_ref, out_ref):
  m = 2 * N
  z = z_ref[...]
  zn = z / (jnp.sqrt(jnp.sum(z * z, axis=1, keepdims=True)) + 1e-10)
  znb = zn.astype(jnp.bfloat16)
  s = lax.dot_general(znb, znb, (((1,), (1,)), ((), ())),
                      preferred_element_type=jnp.float32)
  r = lax.broadcasted_iota(jnp.int32, (m, m), 0)
  c = lax.broadcasted_iota(jnp.int32, (m, m), 1)
  sm = jnp.where(r == c, jnp.float32(-1e30), s * (1.0 / TEMP))
  lse_i = jnp.log(jnp.sum(jnp.exp(sm), axis=1, keepdims=True))
  pos = jnp.sum(jnp.where((r ^ 1) == c, s, 0.0), axis=1, keepdims=True)
  l_ins = jnp.sum(lse_i - pos * (1.0 / TEMP)) / m

  fo_bf = fo_ref[...].astype(jnp.bfloat16)
  lab = lab_ref[...]
  s_tot = jnp.zeros((N, 1), jnp.float32)
  lab_tot = jnp.zeros((N, 1), jnp.float32)
  bounds = [(i * _BK, _BK) for i in range(_NFULL)] + [(_NFULL * _BK, _KT)]
  for lo, width in bounds:
    scores = jnp.dot(fo_bf, hw_ref[:, lo:lo + width],
                     preferred_element_type=jnp.float32) + hb_ref[:, lo:lo + width]
    col = lo + lax.broadcasted_iota(jnp.int32, (N, width), 1)
    labm = col == lab
    lab_tot = lab_tot + jnp.sum(jnp.where(labm, scores, 0.0),
                                axis=1, keepdims=True)
    s_tot = s_tot + jnp.sum(jnp.exp(scores), axis=1, keepdims=True)

  nll = jnp.log(s_tot) - lab_tot
  wv = w_ref[...]
  l_cls = jnp.sum(wv * nll) / jnp.sum(wv)
  out_ref[...] = jnp.reshape(l_ins + l_cls, (1, 1))


def _tc1(z, f_odc, labels, w, hw_bf, hb2, interpret=False):
  return pl.pallas_call(
      _tc1_body,
      out_shape=jax.ShapeDtypeStruct((1, 1), jnp.float32),
      interpret=interpret,
  )(z, f_odc, labels, w, hw_bf, hb2)


_CCH = 128
_CG = N // _CCH


def _tc2_body(fo_ref, fc_ref, pos_ref, neg_ref, p1_ref, out_ref, acc_sc):
  pid = pl.program_id(0)

  @pl.when(pid == 0)
  def _():
    acc_sc[...] = jnp.zeros((1, 1), jnp.float32)

  fo = fo_ref[...]
  fc = fc_ref[...]
  pc = pos_ref[...]
  fon = fo / jnp.sqrt(jnp.sum(fo * fo, axis=1, keepdims=True))
  fcn = fc / jnp.sqrt(jnp.sum(fc * fc, axis=1, keepdims=True))
  pcn = pc / jnp.sqrt(jnp.sum(pc * pc, axis=1, keepdims=True))
  p1 = jnp.sum(fon * pcn, axis=1, keepdims=True) * (1.0 / TEMP)
  p2 = jnp.sum(fcn * pcn, axis=1, keepdims=True) * (1.0 / TEMP)

  ncg = jnp.reshape(neg_ref[...], (_CCH, KN, D))
  ncb = ncg.astype(jnp.bfloat16)
  nss = jnp.sum(ncg * ncg, axis=2)
  inv = lax.rsqrt(nss)
  fon_b = fon.astype(jnp.bfloat16)
  fcn_b = fcn.astype(jnp.bfloat16)
  d1 = jnp.sum((fon_b[:, None, :] * ncb).astype(jnp.float32),
               axis=2) * inv * (1.0 / TEMP)
  d2 = jnp.sum((fcn_b[:, None, :] * ncb).astype(jnp.float32),
               axis=2) * inv * (1.0 / TEMP)
  lse1 = jnp.log(jnp.exp(p1) + jnp.sum(jnp.exp(d1), axis=1, keepdims=True))
  lse2 = jnp.log(jnp.exp(p2) + jnp.sum(jnp.exp(d2), axis=1, keepdims=True))
  acc_sc[...] = acc_sc[...] + jnp.reshape(
      jnp.sum(lse1 - p1) + jnp.sum(lse2 - p2), (1, 1))

  @pl.when(pid == _CG - 1)
  def _():
    out_ref[...] = p1_ref[...] + acc_sc[...] * (1.0 / N)


def _tc2(f_odc, f_cts, pos_c, neg_c, part1, interpret=False):
  return pl.pallas_call(
      _tc2_body,
      grid=(_CG,),
      in_specs=[
          pl.BlockSpec((_CCH, D), lambda i: (i, 0)),
          pl.BlockSpec((_CCH, D), lambda i: (i, 0)),
          pl.BlockSpec((_CCH, D), lambda i: (i, 0)),
          pl.BlockSpec((_CCH * KN, D), lambda i: (i, 0)),
          pl.BlockSpec((1, 1), lambda i: (0, 0)),
      ],
      out_specs=pl.BlockSpec((1, 1), lambda i: (0, 0)),
      out_shape=jax.ShapeDtypeStruct((1, 1), jnp.float32),
      scratch_shapes=[
          pltpu.VMEM((1, 1), jnp.float32),
      ],
      interpret=interpret,
  )(f_odc, f_cts, pos_c, neg_c, part1)


def kernel(z, idx, neg_idx, centroids, label_bank, head_w, head_b, loss_weight):
  labels, w, pos_c = _sc_meta(
      idx.astype(jnp.int32), centroids, label_bank.astype(jnp.int32),
      loss_weight)
  neg_c = _sc_neg(neg_idx.reshape(-1).astype(jnp.int32), centroids)
  f_odc = z[0::2]
  f_cts = z[1::2]
  hw_bf = head_w.astype(jnp.bfloat16)
  hb2 = head_b.reshape(1, K)
  part1 = _tc1(z, f_odc, labels.reshape(N, 1), w.reshape(N, 1), hw_bf, hb2)
  out = _tc2(f_odc, f_cts, pos_c, neg_c, part1)
  return out[0, 0]
